# baseline (speedup 1.0000x reference)
import functools

import jax
import jax.numpy as jnp
from jax import lax
from jax.experimental import pallas as pl
from jax.experimental.pallas import tpu as pltpu
from jax.experimental.pallas import tpu_sc as plsc


def _prep_body(H, W, flow_ref, metric_ref, t_ref, wm_ref):
    i = pl.program_id(1)
    x = lax.broadcasted_iota(jnp.int32, (8, W), 1).astype(jnp.float32)
    y = (lax.broadcasted_iota(jnp.int32, (8, W), 0) + i * 8).astype(jnp.float32)
    fx = x + flow_ref[0, 0]
    fy = y + flow_ref[0, 1]
    x0f = jnp.floor(fx)
    y0f = jnp.floor(fy)
    x1f = x0f + 1.0
    y1f = y0f + 1.0
    m = jnp.exp(metric_ref[0, 0])
    wnw = (x1f - fx) * (y1f - fy)
    wne = (fx - x0f) * (y1f - fy)
    wsw = (x1f - fx) * (fy - y0f)
    wse = (fx - x0f) * (fy - y0f)
    mx0 = (x0f >= 0.0) & (x0f <= W - 1.0)
    mx1 = (x1f >= 0.0) & (x1f <= W - 1.0)
    my0 = (y0f >= 0.0) & (y0f <= H - 1.0)
    my1 = (y1f >= 0.0) & (y1f <= H - 1.0)
    zero = jnp.float32(0.0)
    wm_ref[0, 0] = jnp.where(mx0 & my0, wnw, zero) * m
    wm_ref[1, 0] = jnp.where(mx1 & my0, wne, zero) * m
    wm_ref[2, 0] = jnp.where(mx0 & my1, wsw, zero) * m
    wm_ref[3, 0] = jnp.where(mx1 & my1, wse, zero) * m
    ex = jnp.clip(x0f, -1.0, W - 1.0) + 1.0
    ey = jnp.clip(y0f, -1.0, H - 1.0) + 1.0
    t_ref[0] = (ey * W + ex).astype(jnp.int32)


def _make_splat(B, C, H, W, P, interpret=False):
    HW = H * W
    HALF = HW // 2
    NTASK = 2 * C + 2
    NROUND = (NTASK + 15) // 16
    OFFS = (W + 1, W, 1, 0)
    assert HW % P == 0 and HALF % P == 0 and P % 16 == 0

    NWIN = HW // P
    assert NWIN % 2 == 0

    mesh = plsc.VectorSubcoreMesh(core_axis_name="c", subcore_axis_name="s")

    @functools.partial(
        pl.kernel,
        out_type=(jax.ShapeDtypeStruct((B * C * HW,), jnp.float32),
                  jax.ShapeDtypeStruct((B * HW,), jnp.float32)),
        mesh=mesh,
        scratch_types=[
            pltpu.VMEM((HALF,), jnp.float32),
            pltpu.VMEM((2, P), jnp.float32),
            pltpu.VMEM((2, P), jnp.int32),
            pltpu.VMEM((2, P), jnp.float32),
            pltpu.VMEM((2, P), jnp.float32),
            pltpu.VMEM((2, P), jnp.float32),
            pltpu.VMEM((2, P), jnp.float32),
            pltpu.VMEM((P,), jnp.float32),
            pltpu.VMEM((P,), jnp.float32),
            pltpu.SemaphoreType.DMA,
            pltpu.SemaphoreType.DMA,
        ],
        compiler_params=pltpu.CompilerParams(needs_layout_passes=False),
        interpret=interpret,
    )
    def splat(inp_hbm, t_hbm, wm_hbm, out_hbm, rn_hbm,
              acc, inp_b, t_b, w0, w1, w2, w3, rb, ob, sem0, sem1):
        b = lax.axis_index("c")
        s = lax.axis_index("s")
        wbufs = (w0, w1, w2, w3)
        sems = (sem0, sem1)

        def accumulate(c, h, with_input):
            base = h * HALF

            @plsc.parallel_loop(0, HALF // 16, unroll=8)
            def _(i):
                acc[pl.ds(pl.multiple_of(i * 16, 16), 16)] = jnp.zeros((16,), jnp.float32)

            def issue(wi, pb):
                p0 = wi * P
                if with_input:
                    pltpu.async_copy(
                        inp_hbm.at[pl.ds(pl.multiple_of((b * C + c) * HW + p0, 8), P)],
                        inp_b.at[pb], sems[pb])
                pltpu.async_copy(
                    t_hbm.at[pl.ds(pl.multiple_of(b * HW + p0, 8), P)],
                    t_b.at[pb], sems[pb])
                for k in range(4):
                    pltpu.async_copy(
                        wm_hbm.at[pl.ds(pl.multiple_of((k * B + b) * HW + p0, 8), P)],
                        wbufs[k].at[pb], sems[pb])

            def wait(pb):
                if with_input:
                    pltpu.make_async_copy(
                        inp_hbm.at[pl.ds(0, P)], inp_b.at[pb], sems[pb]).wait()
                pltpu.make_async_copy(
                    t_hbm.at[pl.ds(0, P)], t_b.at[pb], sems[pb]).wait()
                for k in range(4):
                    pltpu.make_async_copy(
                        wm_hbm.at[pl.ds(0, P)], wbufs[k].at[pb], sems[pb]).wait()

            lim = jnp.uint32(HALF)

            def compute(pb):
                @plsc.parallel_loop(0, P // 16, unroll=4)
                def _(g):
                    s16 = pl.ds(pl.multiple_of(g * 16, 16), 16)
                    tv = t_b[pb, s16]
                    iv = inp_b[pb, s16] if with_input else None
                    for k in range(4):
                        wv = wbufs[k][pb, s16]
                        loc = tv - (base + OFFS[k])
                        msk = plsc.bitcast(loc, jnp.uint32) < lim
                        val = iv * wv if with_input else wv
                        plsc.addupdate_scatter(acc, [loc], val, mask=msk)

            issue(0, 0)
            issue(1, 1)

            def win2(j, _):
                for pb in range(2):
                    wi = j * 2 + pb
                    wait(pb)
                    compute(pb)

                    @pl.when(wi + 2 < NWIN)
                    def _():
                        issue(wi + 2, pb)
                return 0
            lax.fori_loop(0, NWIN // 2, win2, 0)

        def flush(c, h):
            def chunk(qi, _):
                q0 = qi * P
                pltpu.sync_copy(
                    rn_hbm.at[pl.ds(pl.multiple_of(b * HW + h * HALF + q0, 8), P)], rb)

                @plsc.parallel_loop(0, P // 16, unroll=4)
                def _(g):
                    s16 = pl.ds(pl.multiple_of(g * 16, 16), 16)
                    a16 = pl.ds(pl.multiple_of(q0 + g * 16, 16), 16)
                    ob[s16] = acc[a16] * rb[s16]
                pltpu.sync_copy(
                    ob,
                    out_hbm.at[pl.ds(pl.multiple_of((b * C + c) * HW + h * HALF + q0, 8), P)])
                return 0
            lax.fori_loop(0, HALF // P, chunk, 0)

        def norm_finalize(h):
            def chunk(qi, _):
                q0 = qi * P

                @plsc.parallel_loop(0, P // 16, unroll=4)
                def _(g):
                    s16 = pl.ds(pl.multiple_of(g * 16, 16), 16)
                    a16 = pl.ds(pl.multiple_of(q0 + g * 16, 16), 16)
                    n = acc[a16]
                    d = jnp.where(n == 0.0, jnp.float32(1.0), n)
                    rb[s16] = jnp.float32(1.0) / d
                pltpu.sync_copy(
                    rb, rn_hbm.at[pl.ds(pl.multiple_of(b * HW + h * HALF + q0, 8), P)])
                return 0
            lax.fori_loop(0, HALF // P, chunk, 0)

        ct0 = s - 2
        c0 = ct0 // 2
        h0 = ct0 % 2

        @pl.when(s < 2)
        def _():
            accumulate(0, s, with_input=False)
            norm_finalize(s)

        @pl.when(s >= 2)
        def _():
            accumulate(c0, h0, with_input=True)

        plsc.subcore_barrier()

        @pl.when(s >= 2)
        def _():
            flush(c0, h0)

        def round_body(r, _):
            task = r * 16 + s
            ct = task - 2
            c = ct // 2
            h = ct % 2

            @pl.when(task < NTASK)
            def _():
                accumulate(c, h, with_input=True)
                flush(c, h)
            return 0
        lax.fori_loop(1, NROUND, round_body, 0)

    return splat


def kernel(tenInput, tenFlow, tenMetric):
    B, C, H, W = tenInput.shape
    HW = H * W

    t, wm = pl.pallas_call(
        functools.partial(_prep_body, H, W),
        grid=(B, H // 8),
        in_specs=[
            pl.BlockSpec((1, 2, 8, W), lambda b, i: (b, 0, i, 0)),
            pl.BlockSpec((1, 1, 8, W), lambda b, i: (b, 0, i, 0)),
        ],
        out_specs=[
            pl.BlockSpec((1, 8, W), lambda b, i: (b, i, 0)),
            pl.BlockSpec((4, 1, 8, W), lambda b, i: (0, b, i, 0)),
        ],
        out_shape=[
            jax.ShapeDtypeStruct((B, H, W), jnp.int32),
            jax.ShapeDtypeStruct((4, B, H, W), jnp.float32),
        ],
    )(tenFlow, tenMetric)

    splat = _make_splat(B, C, H, W, P=2048)
    out_flat, _ = splat(tenInput.reshape(B * C * HW),
                        t.reshape(B * HW),
                        wm.reshape(4 * B * HW))
    return out_flat.reshape(B, C, H, W)


# speedup vs baseline: 1.2745x; 1.2745x over previous
import functools

import jax
import jax.numpy as jnp
from jax import lax
from jax.experimental import pallas as pl
from jax.experimental.pallas import tpu as pltpu
from jax.experimental.pallas import tpu_sc as plsc


def _prep_body(H, W, flow_ref, metric_ref, t_ref, wm_ref):
    i = pl.program_id(1)
    x = lax.broadcasted_iota(jnp.int32, (8, W), 1).astype(jnp.float32)
    y = (lax.broadcasted_iota(jnp.int32, (8, W), 0) + i * 8).astype(jnp.float32)
    fx = x + flow_ref[0, 0]
    fy = y + flow_ref[0, 1]
    x0f = jnp.floor(fx)
    y0f = jnp.floor(fy)
    x1f = x0f + 1.0
    y1f = y0f + 1.0
    m = jnp.exp(metric_ref[0, 0])
    wnw = (x1f - fx) * (y1f - fy)
    wne = (fx - x0f) * (y1f - fy)
    wsw = (x1f - fx) * (fy - y0f)
    wse = (fx - x0f) * (fy - y0f)
    mx0 = (x0f >= 0.0) & (x0f <= W - 1.0)
    mx1 = (x1f >= 0.0) & (x1f <= W - 1.0)
    my0 = (y0f >= 0.0) & (y0f <= H - 1.0)
    my1 = (y1f >= 0.0) & (y1f <= H - 1.0)
    zero = jnp.float32(0.0)
    wm_ref[0, 0] = jnp.where(mx0 & my0, wnw, zero) * m
    wm_ref[1, 0] = jnp.where(mx1 & my0, wne, zero) * m
    wm_ref[2, 0] = jnp.where(mx0 & my1, wsw, zero) * m
    wm_ref[3, 0] = jnp.where(mx1 & my1, wse, zero) * m
    ex = jnp.clip(x0f, -1.0, W - 1.0) + 1.0
    ey = jnp.clip(y0f, -1.0, H - 1.0) + 1.0
    t_ref[0] = (ey * W + ex).astype(jnp.int32)


def _flag_body(H, W, t_ref, flag_ref):
    B = t_ref.shape[0]
    HW = H * W
    HALF = HW // 2
    NW = H // 8
    t = t_ref[...].reshape(B * NW, 8 * W)
    tmin = jnp.min(t, axis=1)
    tmax = jnp.max(t, axis=1)
    cols = []
    for h in (0, 1):
        hit = (tmax >= h * HALF) & (tmin <= h * HALF + HALF + W)
        cols.append(hit.astype(jnp.int32))
    flag_ref[...] = jnp.stack(cols, axis=1).reshape(B, NW, 2)


def _make_splat(B, C, H, W, P, interpret=False):
    HW = H * W
    HALF = HW // 2
    assert P == 8 * W
    NTASK = 2 * C + 2
    NROUND = (NTASK + 15) // 16
    OFFS = (W + 1, W, 1, 0)
    assert HW % P == 0 and HALF % P == 0 and P % 16 == 0

    NWIN = HW // P
    assert NWIN % 2 == 0

    mesh = plsc.VectorSubcoreMesh(core_axis_name="c", subcore_axis_name="s")

    @functools.partial(
        pl.kernel,
        out_type=(jax.ShapeDtypeStruct((B * C * HW,), jnp.float32),
                  jax.ShapeDtypeStruct((B * HW,), jnp.float32)),
        mesh=mesh,
        scratch_types=[
            pltpu.VMEM((HALF,), jnp.float32),
            pltpu.VMEM((2, P), jnp.float32),
            pltpu.VMEM((2, P), jnp.int32),
            pltpu.VMEM((2, P), jnp.float32),
            pltpu.VMEM((2, P), jnp.float32),
            pltpu.VMEM((2, P), jnp.float32),
            pltpu.VMEM((2, P), jnp.float32),
            pltpu.VMEM((P,), jnp.float32),
            pltpu.VMEM((P,), jnp.float32),
            pltpu.VMEM((2 * HW // P, ), jnp.int32),
            pltpu.SemaphoreType.DMA,
            pltpu.SemaphoreType.DMA,
        ],
        compiler_params=pltpu.CompilerParams(needs_layout_passes=False),
        interpret=interpret,
    )
    def splat(inp_hbm, t_hbm, wm_hbm, flags_hbm, out_hbm, rn_hbm,
              acc, inp_b, t_b, w0, w1, w2, w3, rb, ob, flag_buf, sem0, sem1):
        b = lax.axis_index("c")
        s = lax.axis_index("s")
        wbufs = (w0, w1, w2, w3)
        sems = (sem0, sem1)

        def accumulate(c, h, with_input):
            base = h * HALF
            pltpu.sync_copy(
                flags_hbm.at[pl.ds(pl.multiple_of(b * 2 * NWIN, 8), 2 * NWIN)],
                flag_buf)

            def wflag(wi):
                gat = plsc.load_gather(
                    flag_buf, [jnp.full((16,), wi * 2 + h, jnp.int32)])
                return jnp.max(gat) > 0

            @plsc.parallel_loop(0, HALF // 16, unroll=8)
            def _(i):
                acc[pl.ds(pl.multiple_of(i * 16, 16), 16)] = jnp.zeros((16,), jnp.float32)

            def issue(wi, pb):
                p0 = wi * P
                if with_input:
                    pltpu.async_copy(
                        inp_hbm.at[pl.ds(pl.multiple_of((b * C + c) * HW + p0, 8), P)],
                        inp_b.at[pb], sems[pb])
                pltpu.async_copy(
                    t_hbm.at[pl.ds(pl.multiple_of(b * HW + p0, 8), P)],
                    t_b.at[pb], sems[pb])
                for k in range(4):
                    pltpu.async_copy(
                        wm_hbm.at[pl.ds(pl.multiple_of((k * B + b) * HW + p0, 8), P)],
                        wbufs[k].at[pb], sems[pb])

            def wait(pb):
                if with_input:
                    pltpu.make_async_copy(
                        inp_hbm.at[pl.ds(0, P)], inp_b.at[pb], sems[pb]).wait()
                pltpu.make_async_copy(
                    t_hbm.at[pl.ds(0, P)], t_b.at[pb], sems[pb]).wait()
                for k in range(4):
                    pltpu.make_async_copy(
                        wm_hbm.at[pl.ds(0, P)], wbufs[k].at[pb], sems[pb]).wait()

            lim = jnp.uint32(HALF)

            def compute(pb):
                @plsc.parallel_loop(0, P // 16, unroll=4)
                def _(g):
                    s16 = pl.ds(pl.multiple_of(g * 16, 16), 16)
                    tv = t_b[pb, s16]
                    iv = inp_b[pb, s16] if with_input else None
                    for k in range(4):
                        wv = wbufs[k][pb, s16]
                        loc = tv - (base + OFFS[k])
                        msk = plsc.bitcast(loc, jnp.uint32) < lim
                        val = iv * wv if with_input else wv
                        plsc.addupdate_scatter(acc, [loc], val, mask=msk)

            for pb in range(2):
                @pl.when(wflag(pb))
                def _():
                    issue(pb, pb)

            def win2(j, _):
                for pb in range(2):
                    wi = j * 2 + pb

                    @pl.when(wflag(wi))
                    def _():
                        wait(pb)
                        compute(pb)

                    @pl.when((wi + 2 < NWIN) & wflag(wi + 2))
                    def _():
                        issue(wi + 2, pb)
                return 0
            lax.fori_loop(0, NWIN // 2, win2, 0)

        def flush(c, h):
            def chunk(qi, _):
                q0 = qi * P
                pltpu.sync_copy(
                    rn_hbm.at[pl.ds(pl.multiple_of(b * HW + h * HALF + q0, 8), P)], rb)

                @plsc.parallel_loop(0, P // 16, unroll=4)
                def _(g):
                    s16 = pl.ds(pl.multiple_of(g * 16, 16), 16)
                    a16 = pl.ds(pl.multiple_of(q0 + g * 16, 16), 16)
                    ob[s16] = acc[a16] * rb[s16]
                pltpu.sync_copy(
                    ob,
                    out_hbm.at[pl.ds(pl.multiple_of((b * C + c) * HW + h * HALF + q0, 8), P)])
                return 0
            lax.fori_loop(0, HALF // P, chunk, 0)

        def norm_finalize(h):
            def chunk(qi, _):
                q0 = qi * P

                @plsc.parallel_loop(0, P // 16, unroll=4)
                def _(g):
                    s16 = pl.ds(pl.multiple_of(g * 16, 16), 16)
                    a16 = pl.ds(pl.multiple_of(q0 + g * 16, 16), 16)
                    n = acc[a16]
                    d = jnp.where(n == 0.0, jnp.float32(1.0), n)
                    rb[s16] = jnp.float32(1.0) / d
                pltpu.sync_copy(
                    rb, rn_hbm.at[pl.ds(pl.multiple_of(b * HW + h * HALF + q0, 8), P)])
                return 0
            lax.fori_loop(0, HALF // P, chunk, 0)

        ct0 = s - 2
        c0 = ct0 // 2
        h0 = ct0 % 2

        @pl.when(s < 2)
        def _():
            accumulate(0, s, with_input=False)
            norm_finalize(s)

        @pl.when(s >= 2)
        def _():
            accumulate(c0, h0, with_input=True)

        plsc.subcore_barrier()

        @pl.when(s >= 2)
        def _():
            flush(c0, h0)

        def round_body(r, _):
            task = r * 16 + s
            ct = task - 2
            c = ct // 2
            h = ct % 2

            @pl.when(task < NTASK)
            def _():
                accumulate(c, h, with_input=True)
                flush(c, h)
            return 0
        lax.fori_loop(1, NROUND, round_body, 0)

    return splat


def kernel(tenInput, tenFlow, tenMetric):
    B, C, H, W = tenInput.shape
    HW = H * W

    t, wm = pl.pallas_call(
        functools.partial(_prep_body, H, W),
        grid=(B, H // 8),
        in_specs=[
            pl.BlockSpec((1, 2, 8, W), lambda b, i: (b, 0, i, 0)),
            pl.BlockSpec((1, 1, 8, W), lambda b, i: (b, 0, i, 0)),
        ],
        out_specs=[
            pl.BlockSpec((1, 8, W), lambda b, i: (b, i, 0)),
            pl.BlockSpec((4, 1, 8, W), lambda b, i: (0, b, i, 0)),
        ],
        out_shape=[
            jax.ShapeDtypeStruct((B, H, W), jnp.int32),
            jax.ShapeDtypeStruct((4, B, H, W), jnp.float32),
        ],
    )(tenFlow, tenMetric)

    flags = pl.pallas_call(
        functools.partial(_flag_body, H, W),
        out_shape=jax.ShapeDtypeStruct((B, H // 8, 2), jnp.int32),
    )(t)

    splat = _make_splat(B, C, H, W, P=8 * W)
    out_flat, _ = splat(tenInput.reshape(B * C * HW),
                        t.reshape(B * HW),
                        wm.reshape(4 * B * HW),
                        flags.reshape(B * (H // 8) * 2))
    return out_flat.reshape(B, C, H, W)


# speedup vs baseline: 1.3528x; 1.0614x over previous
import functools

import jax
import jax.numpy as jnp
from jax import lax
from jax.experimental import pallas as pl
from jax.experimental.pallas import tpu as pltpu
from jax.experimental.pallas import tpu_sc as plsc


def _prep_body(H, W, flow_ref, metric_ref, t_ref, wm_ref):
    i = pl.program_id(1)
    x = lax.broadcasted_iota(jnp.int32, (8, W), 1).astype(jnp.float32)
    y = (lax.broadcasted_iota(jnp.int32, (8, W), 0) + i * 8).astype(jnp.float32)
    fx = x + flow_ref[0, 0]
    fy = y + flow_ref[0, 1]
    x0f = jnp.floor(fx)
    y0f = jnp.floor(fy)
    x1f = x0f + 1.0
    y1f = y0f + 1.0
    m = jnp.exp(metric_ref[0, 0])
    wnw = (x1f - fx) * (y1f - fy)
    wne = (fx - x0f) * (y1f - fy)
    wsw = (x1f - fx) * (fy - y0f)
    wse = (fx - x0f) * (fy - y0f)
    mx0 = (x0f >= 0.0) & (x0f <= W - 1.0)
    mx1 = (x1f >= 0.0) & (x1f <= W - 1.0)
    my0 = (y0f >= 0.0) & (y0f <= H - 1.0)
    my1 = (y1f >= 0.0) & (y1f <= H - 1.0)
    zero = jnp.float32(0.0)
    wm_ref[0, 0] = jnp.where(mx0 & my0, wnw, zero) * m
    wm_ref[1, 0] = jnp.where(mx1 & my0, wne, zero) * m
    wm_ref[2, 0] = jnp.where(mx0 & my1, wsw, zero) * m
    wm_ref[3, 0] = jnp.where(mx1 & my1, wse, zero) * m
    ex = jnp.clip(x0f, -1.0, W - 1.0) + 1.0
    ey = jnp.clip(y0f, -1.0, H - 1.0) + 1.0
    t_ref[0] = (ey * W + ex).astype(jnp.int32)


def _flag_body(H, W, t_ref, flag_ref):
    B = t_ref.shape[0]
    HW = H * W
    HALF = HW // 2
    NW = H // 8
    t = t_ref[...].reshape(B * NW, 8 * W)
    tmin = jnp.min(t, axis=1)
    tmax = jnp.max(t, axis=1)
    cols = []
    for h in (0, 1):
        hit = (tmax >= h * HALF) & (tmin <= h * HALF + HALF + W)
        cols.append(hit.astype(jnp.int32))
    flag_ref[...] = jnp.stack(cols, axis=1).reshape(B, NW, 2)


def _make_splat(B, C, H, W, P, interpret=False):
    HW = H * W
    HALF = HW // 2
    assert P == 8 * W
    NTASK = 2 * C + 2
    NROUND = (NTASK + 15) // 16
    OFFS = (W + 1, W, 1, 0)
    assert HW % P == 0 and HALF % P == 0 and P % 16 == 0

    NWIN = HW // P
    assert NWIN % 2 == 0

    mesh = plsc.VectorSubcoreMesh(core_axis_name="c", subcore_axis_name="s")

    @functools.partial(
        pl.kernel,
        out_type=(jax.ShapeDtypeStruct((B * C * HW,), jnp.float32),
                  jax.ShapeDtypeStruct((B * HW,), jnp.float32)),
        mesh=mesh,
        scratch_types=[
            pltpu.VMEM((HALF,), jnp.float32),
            pltpu.VMEM((2, P), jnp.float32),
            pltpu.VMEM((2, P), jnp.int32),
            pltpu.VMEM((2, P), jnp.float32),
            pltpu.VMEM((2, P), jnp.float32),
            pltpu.VMEM((2, P), jnp.float32),
            pltpu.VMEM((2, P), jnp.float32),
            pltpu.VMEM((2, P), jnp.float32),
            pltpu.VMEM((2, P), jnp.float32),
            pltpu.VMEM((2 * HW // P, ), jnp.int32),
            pltpu.SemaphoreType.DMA,
            pltpu.SemaphoreType.DMA,
            pltpu.SemaphoreType.DMA,
            pltpu.SemaphoreType.DMA,
        ],
        compiler_params=pltpu.CompilerParams(needs_layout_passes=False),
        interpret=interpret,
    )
    def splat(inp_hbm, t_hbm, wm_hbm, flags_hbm, out_hbm, rn_hbm,
              acc, inp_b, t_b, w0, w1, w2, w3, rb, ob, flag_buf,
              sem0, sem1, semo0, semo1):
        b = lax.axis_index("c")
        s = lax.axis_index("s")
        wbufs = (w0, w1, w2, w3)
        sems = (sem0, sem1)
        semo = (semo0, semo1)
        pltpu.sync_copy(
            flags_hbm.at[pl.ds(pl.multiple_of(b * 2 * NWIN, 8), 2 * NWIN)],
            flag_buf)

        def accumulate(c, h, with_input):
            base = h * HALF

            def wflag(wi):
                gat = plsc.load_gather(
                    flag_buf, [jnp.full((16,), wi * 2 + h, jnp.int32)])
                return jnp.max(gat) > 0

            @plsc.parallel_loop(0, HALF // 16, unroll=8)
            def _(i):
                acc[pl.ds(pl.multiple_of(i * 16, 16), 16)] = jnp.zeros((16,), jnp.float32)

            def issue(wi, pb):
                p0 = wi * P
                if with_input:
                    pltpu.async_copy(
                        inp_hbm.at[pl.ds(pl.multiple_of((b * C + c) * HW + p0, 8), P)],
                        inp_b.at[pb], sems[pb])
                pltpu.async_copy(
                    t_hbm.at[pl.ds(pl.multiple_of(b * HW + p0, 8), P)],
                    t_b.at[pb], sems[pb])
                for k in range(4):
                    pltpu.async_copy(
                        wm_hbm.at[pl.ds(pl.multiple_of((k * B + b) * HW + p0, 8), P)],
                        wbufs[k].at[pb], sems[pb])

            def wait(pb):
                if with_input:
                    pltpu.make_async_copy(
                        inp_hbm.at[pl.ds(0, P)], inp_b.at[pb], sems[pb]).wait()
                pltpu.make_async_copy(
                    t_hbm.at[pl.ds(0, P)], t_b.at[pb], sems[pb]).wait()
                for k in range(4):
                    pltpu.make_async_copy(
                        wm_hbm.at[pl.ds(0, P)], wbufs[k].at[pb], sems[pb]).wait()

            lim = jnp.uint32(HALF)

            def compute(pb):
                @plsc.parallel_loop(0, P // 16, unroll=8)
                def _(g):
                    s16 = pl.ds(pl.multiple_of(g * 16, 16), 16)
                    tv = t_b[pb, s16]
                    iv = inp_b[pb, s16] if with_input else None
                    for k in range(4):
                        wv = wbufs[k][pb, s16]
                        loc = tv - (base + OFFS[k])
                        msk = plsc.bitcast(loc, jnp.uint32) < lim
                        val = iv * wv if with_input else wv
                        plsc.addupdate_scatter(acc, [loc], val, mask=msk)

            for pb in range(2):
                @pl.when(wflag(pb))
                def _():
                    issue(pb, pb)

            def win2(j, _):
                for pb in range(2):
                    wi = j * 2 + pb

                    @pl.when(wflag(wi))
                    def _():
                        wait(pb)
                        compute(pb)

                    @pl.when((wi + 2 < NWIN) & wflag(wi + 2))
                    def _():
                        issue(wi + 2, pb)
                return 0
            lax.fori_loop(0, NWIN // 2, win2, 0)

        def flush(c, h):
            NCH = HALF // P

            def rb_issue(qi, pb):
                pltpu.async_copy(
                    rn_hbm.at[pl.ds(pl.multiple_of(b * HW + h * HALF + qi * P, 8), P)],
                    rb.at[pb], sems[pb])

            rb_issue(0, 0)
            rb_issue(1, 1)

            def chunk2(j, _):
                for pb in range(2):
                    qi = j * 2 + pb
                    q0 = qi * P
                    pltpu.make_async_copy(
                        rn_hbm.at[pl.ds(0, P)], rb.at[pb], sems[pb]).wait()

                    @pl.when(qi + 2 < NCH)
                    def _():
                        rb_issue(qi + 2, pb)

                    @pl.when(qi >= 2)
                    def _():
                        pltpu.make_async_copy(
                            ob.at[pb],
                            out_hbm.at[pl.ds(0, P)], semo[pb]).wait()

                    @plsc.parallel_loop(0, P // 16, unroll=4)
                    def _(g):
                        s16 = pl.ds(pl.multiple_of(g * 16, 16), 16)
                        a16 = pl.ds(pl.multiple_of(q0 + g * 16, 16), 16)
                        ob[pb, s16] = acc[a16] * rb[pb, s16]
                    pltpu.async_copy(
                        ob.at[pb],
                        out_hbm.at[pl.ds(pl.multiple_of(
                            (b * C + c) * HW + h * HALF + q0, 8), P)],
                        semo[pb])
                return 0
            lax.fori_loop(0, NCH // 2, chunk2, 0)
            for pb in range(2):
                pltpu.make_async_copy(
                    ob.at[pb], out_hbm.at[pl.ds(0, P)], semo[pb]).wait()

        def norm_finalize(h):
            NCH = HALF // P

            def chunk2(j, _):
                for pb in range(2):
                    qi = j * 2 + pb
                    q0 = qi * P

                    @pl.when(qi >= 2)
                    def _():
                        pltpu.make_async_copy(
                            ob.at[pb], rn_hbm.at[pl.ds(0, P)], semo[pb]).wait()

                    @plsc.parallel_loop(0, P // 16, unroll=4)
                    def _(g):
                        s16 = pl.ds(pl.multiple_of(g * 16, 16), 16)
                        a16 = pl.ds(pl.multiple_of(q0 + g * 16, 16), 16)
                        n = acc[a16]
                        d = jnp.where(n == 0.0, jnp.float32(1.0), n)
                        ob[pb, s16] = jnp.float32(1.0) / d
                    pltpu.async_copy(
                        ob.at[pb],
                        rn_hbm.at[pl.ds(pl.multiple_of(b * HW + h * HALF + q0, 8), P)],
                        semo[pb])
                return 0
            lax.fori_loop(0, NCH // 2, chunk2, 0)
            for pb in range(2):
                pltpu.make_async_copy(
                    ob.at[pb], rn_hbm.at[pl.ds(0, P)], semo[pb]).wait()

        ct0 = s - 2
        c0 = ct0 // 2
        h0 = ct0 % 2

        @pl.when(s < 2)
        def _():
            accumulate(0, s, with_input=False)
            norm_finalize(s)

        @pl.when(s >= 2)
        def _():
            accumulate(c0, h0, with_input=True)

        plsc.subcore_barrier()

        @pl.when(s >= 2)
        def _():
            flush(c0, h0)

        def round_body(r, _):
            task = r * 16 + s
            ct = task - 2
            c = ct // 2
            h = ct % 2

            @pl.when(task < NTASK)
            def _():
                accumulate(c, h, with_input=True)
                flush(c, h)
            return 0
        lax.fori_loop(1, NROUND, round_body, 0)

    return splat


def kernel(tenInput, tenFlow, tenMetric):
    B, C, H, W = tenInput.shape
    HW = H * W

    t, wm = pl.pallas_call(
        functools.partial(_prep_body, H, W),
        grid=(B, H // 8),
        in_specs=[
            pl.BlockSpec((1, 2, 8, W), lambda b, i: (b, 0, i, 0)),
            pl.BlockSpec((1, 1, 8, W), lambda b, i: (b, 0, i, 0)),
        ],
        out_specs=[
            pl.BlockSpec((1, 8, W), lambda b, i: (b, i, 0)),
            pl.BlockSpec((4, 1, 8, W), lambda b, i: (0, b, i, 0)),
        ],
        out_shape=[
            jax.ShapeDtypeStruct((B, H, W), jnp.int32),
            jax.ShapeDtypeStruct((4, B, H, W), jnp.float32),
        ],
    )(tenFlow, tenMetric)

    flags = pl.pallas_call(
        functools.partial(_flag_body, H, W),
        out_shape=jax.ShapeDtypeStruct((B, H // 8, 2), jnp.int32),
    )(t)

    splat = _make_splat(B, C, H, W, P=8 * W)
    out_flat, _ = splat(tenInput.reshape(B * C * HW),
                        t.reshape(B * HW),
                        wm.reshape(4 * B * HW),
                        flags.reshape(B * (H // 8) * 2))
    return out_flat.reshape(B, C, H, W)


# speedup vs baseline: 1.5677x; 1.1588x over previous
import functools

import jax
import jax.numpy as jnp
from jax import lax
from jax.experimental import pallas as pl
from jax.experimental.pallas import tpu as pltpu
from jax.experimental.pallas import tpu_sc as plsc


def _prep_body(H, W, flow_ref, metric_ref, t_ref, wm_ref):
    i = pl.program_id(1)
    x = lax.broadcasted_iota(jnp.int32, (8, W), 1).astype(jnp.float32)
    y = (lax.broadcasted_iota(jnp.int32, (8, W), 0) + i * 8).astype(jnp.float32)
    fx = x + flow_ref[0, 0]
    fy = y + flow_ref[0, 1]
    x0f = jnp.floor(fx)
    y0f = jnp.floor(fy)
    x1f = x0f + 1.0
    y1f = y0f + 1.0
    m = jnp.exp(metric_ref[0, 0])
    wnw = (x1f - fx) * (y1f - fy)
    wne = (fx - x0f) * (y1f - fy)
    wsw = (x1f - fx) * (fy - y0f)
    wse = (fx - x0f) * (fy - y0f)
    mx0 = (x0f >= 0.0) & (x0f <= W - 1.0)
    mx1 = (x1f >= 0.0) & (x1f <= W - 1.0)
    my0 = (y0f >= 0.0) & (y0f <= H - 1.0)
    my1 = (y1f >= 0.0) & (y1f <= H - 1.0)
    zero = jnp.float32(0.0)
    wm_ref[0, 0] = jnp.where(mx0 & my0, wnw, zero) * m
    wm_ref[1, 0] = jnp.where(mx1 & my0, wne, zero) * m
    wm_ref[2, 0] = jnp.where(mx0 & my1, wsw, zero) * m
    wm_ref[3, 0] = jnp.where(mx1 & my1, wse, zero) * m
    ex = jnp.clip(x0f, -1.0, W - 1.0) + 1.0
    ey = jnp.clip(y0f, -1.0, H - 1.0) + 1.0
    t_ref[0] = (ey * W + ex).astype(jnp.int32)


def _flag_body(H, W, NSEG, t_ref, flag_ref):
    B = t_ref.shape[0]
    HW = H * W
    SEG = HW // NSEG
    NW = H // 8
    t = t_ref[...].reshape(B * NW, 8 * W)
    tmin = jnp.min(t, axis=1)
    tmax = jnp.max(t, axis=1)
    cols = []
    for h in range(NSEG):
        hit = (tmax >= h * SEG) & (tmin <= h * SEG + SEG + W)
        cols.append(hit.astype(jnp.int32))
    flag_ref[...] = jnp.stack(cols, axis=1).reshape(B, NW, NSEG)


def _make_splat(B, C, H, W, interpret=False):
    HW = H * W
    P = 8 * W
    QTR = HW // 4
    NWIN = HW // P
    NTASK = 4 + 4 * (C // 2)
    NROUND = (NTASK + 15) // 16
    OFFS = (W + 1, W, 1, 0)
    assert C % 2 == 0 and HW % P == 0 and QTR % P == 0 and NWIN % 2 == 0

    mesh = plsc.VectorSubcoreMesh(core_axis_name="c", subcore_axis_name="s")

    @functools.partial(
        pl.kernel,
        out_type=(jax.ShapeDtypeStruct((B * C * HW,), jnp.float32),
                  jax.ShapeDtypeStruct((B * HW,), jnp.float32)),
        mesh=mesh,
        scratch_types=[
            pltpu.VMEM((QTR,), jnp.float32),
            pltpu.VMEM((QTR,), jnp.float32),
            pltpu.VMEM((2, P), jnp.float32),
            pltpu.VMEM((2, P), jnp.float32),
            pltpu.VMEM((2, P), jnp.int32),
            pltpu.VMEM((2, P), jnp.float32),
            pltpu.VMEM((2, P), jnp.float32),
            pltpu.VMEM((2, P), jnp.float32),
            pltpu.VMEM((2, P), jnp.float32),
            pltpu.VMEM((4 * HW // P,), jnp.int32),
            pltpu.SemaphoreType.DMA,
            pltpu.SemaphoreType.DMA,
            pltpu.SemaphoreType.DMA,
            pltpu.SemaphoreType.DMA,
        ],
        compiler_params=pltpu.CompilerParams(needs_layout_passes=False),
        interpret=interpret,
    )
    def splat(inp_hbm, t_hbm, wm_hbm, flags_hbm, out_hbm, rn_hbm,
              acc_a, acc_b, inp_a, inp_b, t_b, w0, w1, w2, w3, flag_buf,
              sem0, sem1, semo0, semo1):
        b = lax.axis_index("c")
        s = lax.axis_index("s")
        wbufs = (w0, w1, w2, w3)
        ibufs = (inp_a, inp_b)
        accs = (acc_a, acc_b)
        sems = (sem0, sem1)
        semo = (semo0, semo1)
        pltpu.sync_copy(
            flags_hbm.at[pl.ds(pl.multiple_of(b * 4 * NWIN, 8), 4 * NWIN)],
            flag_buf)

        def wflag(wi, q):
            gat = plsc.load_gather(
                flag_buf, [jnp.full((16,), wi * 4 + q, jnp.int32)])
            return jnp.max(gat) > 0

        def accumulate(c0, q, nch):
            nacc = max(nch, 1)
            base = q * QTR

            for ci in range(nacc):
                @plsc.parallel_loop(0, QTR // 16, unroll=8)
                def _(i):
                    accs[ci][pl.ds(pl.multiple_of(i * 16, 16), 16)] = (
                        jnp.zeros((16,), jnp.float32))

            def issue(wi, pb):
                p0 = wi * P
                for ci in range(nch):
                    pltpu.async_copy(
                        inp_hbm.at[pl.ds(
                            pl.multiple_of((b * C + c0 + ci) * HW + p0, 8), P)],
                        ibufs[ci].at[pb], sems[pb])
                pltpu.async_copy(
                    t_hbm.at[pl.ds(pl.multiple_of(b * HW + p0, 8), P)],
                    t_b.at[pb], sems[pb])
                for k in range(4):
                    pltpu.async_copy(
                        wm_hbm.at[pl.ds(pl.multiple_of((k * B + b) * HW + p0, 8), P)],
                        wbufs[k].at[pb], sems[pb])

            def wait(pb):
                for ci in range(nch):
                    pltpu.make_async_copy(
                        inp_hbm.at[pl.ds(0, P)], ibufs[ci].at[pb], sems[pb]).wait()
                pltpu.make_async_copy(
                    t_hbm.at[pl.ds(0, P)], t_b.at[pb], sems[pb]).wait()
                for k in range(4):
                    pltpu.make_async_copy(
                        wm_hbm.at[pl.ds(0, P)], wbufs[k].at[pb], sems[pb]).wait()

            lim = jnp.uint32(QTR)

            def compute(pb):
                @plsc.parallel_loop(0, P // 16, unroll=8)
                def _(g):
                    s16 = pl.ds(pl.multiple_of(g * 16, 16), 16)
                    tv = t_b[pb, s16]
                    ivs = [ibufs[ci][pb, s16] for ci in range(nch)]
                    for k in range(4):
                        wv = wbufs[k][pb, s16]
                        loc = tv - (base + OFFS[k])
                        msk = plsc.bitcast(loc, jnp.uint32) < lim
                        if nch == 0:
                            plsc.addupdate_scatter(accs[0], [loc], wv, mask=msk)
                        else:
                            for ci in range(nch):
                                plsc.addupdate_scatter(
                                    accs[ci], [loc], ivs[ci] * wv, mask=msk)

            for pb in range(2):
                @pl.when(wflag(pb, q))
                def _():
                    issue(pb, pb)

            def win2(j, _):
                for pb in range(2):
                    wi = j * 2 + pb

                    @pl.when(wflag(wi, q))
                    def _():
                        wait(pb)
                        compute(pb)

                    @pl.when((wi + 2 < NWIN) & wflag(wi + 2, q))
                    def _():
                        issue(wi + 2, pb)
                return 0
            lax.fori_loop(0, NWIN // 2, win2, 0)

        obufs = (w1, w2)

        def flush(c0, q):
            NCH = QTR // P

            def rb_issue(qi, pb):
                pltpu.async_copy(
                    rn_hbm.at[pl.ds(pl.multiple_of(b * HW + q * QTR + qi * P, 8), P)],
                    w0.at[pb], sems[pb])

            rb_issue(0, 0)
            rb_issue(1, 1)

            def chunk2(j, _):
                for pb in range(2):
                    qi = j * 2 + pb
                    q0 = qi * P
                    pltpu.make_async_copy(
                        rn_hbm.at[pl.ds(0, P)], w0.at[pb], sems[pb]).wait()

                    @pl.when(qi >= 2)
                    def _():
                        for ci in range(2):
                            pltpu.make_async_copy(
                                obufs[ci].at[pb],
                                out_hbm.at[pl.ds(0, P)], semo[pb]).wait()

                    for ci in range(2):
                        @plsc.parallel_loop(0, P // 16, unroll=4)
                        def _(g):
                            s16 = pl.ds(pl.multiple_of(g * 16, 16), 16)
                            a16 = pl.ds(pl.multiple_of(q0 + g * 16, 16), 16)
                            obufs[ci][pb, s16] = accs[ci][a16] * w0[pb, s16]
                        pltpu.async_copy(
                            obufs[ci].at[pb],
                            out_hbm.at[pl.ds(pl.multiple_of(
                                (b * C + c0 + ci) * HW + q * QTR + q0, 8), P)],
                            semo[pb])

                    @pl.when(qi + 2 < NCH)
                    def _():
                        rb_issue(qi + 2, pb)
                return 0
            lax.fori_loop(0, NCH // 2, chunk2, 0)
            for pb in range(2):
                for ci in range(2):
                    pltpu.make_async_copy(
                        obufs[ci].at[pb], out_hbm.at[pl.ds(0, P)], semo[pb]).wait()

        def norm_finalize(q):
            NCH = QTR // P

            def chunk2(j, _):
                for pb in range(2):
                    qi = j * 2 + pb
                    q0 = qi * P

                    @pl.when(qi >= 2)
                    def _():
                        pltpu.make_async_copy(
                            w1.at[pb], rn_hbm.at[pl.ds(0, P)], semo[pb]).wait()

                    @plsc.parallel_loop(0, P // 16, unroll=4)
                    def _(g):
                        s16 = pl.ds(pl.multiple_of(g * 16, 16), 16)
                        a16 = pl.ds(pl.multiple_of(q0 + g * 16, 16), 16)
                        n = acc_a[a16]
                        d = jnp.where(n == 0.0, jnp.float32(1.0), n)
                        w1[pb, s16] = jnp.float32(1.0) / d
                    pltpu.async_copy(
                        w1.at[pb],
                        rn_hbm.at[pl.ds(pl.multiple_of(b * HW + q * QTR + q0, 8), P)],
                        semo[pb])
                return 0
            lax.fori_loop(0, NCH // 2, chunk2, 0)
            for pb in range(2):
                pltpu.make_async_copy(
                    w1.at[pb], rn_hbm.at[pl.ds(0, P)], semo[pb]).wait()

        ct0 = s - 4
        c00 = (ct0 // 4) * 2
        q0_ = ct0 % 4

        @pl.when(s < 4)
        def _():
            accumulate(0, s, nch=0)
            norm_finalize(s)

        @pl.when(s >= 4)
        def _():
            accumulate(c00, q0_, nch=2)

        plsc.subcore_barrier()

        @pl.when(s >= 4)
        def _():
            flush(c00, q0_)

        def round_body(r, _):
            task = r * 16 + s
            ct = task - 4
            c0 = (ct // 4) * 2
            q = ct % 4

            @pl.when(task < NTASK)
            def _():
                accumulate(c0, q, nch=2)
                flush(c0, q)
            return 0
        lax.fori_loop(1, NROUND, round_body, 0)

    return splat


def kernel(tenInput, tenFlow, tenMetric):
    B, C, H, W = tenInput.shape
    HW = H * W

    t, wm = pl.pallas_call(
        functools.partial(_prep_body, H, W),
        grid=(B, H // 8),
        in_specs=[
            pl.BlockSpec((1, 2, 8, W), lambda b, i: (b, 0, i, 0)),
            pl.BlockSpec((1, 1, 8, W), lambda b, i: (b, 0, i, 0)),
        ],
        out_specs=[
            pl.BlockSpec((1, 8, W), lambda b, i: (b, i, 0)),
            pl.BlockSpec((4, 1, 8, W), lambda b, i: (0, b, i, 0)),
        ],
        out_shape=[
            jax.ShapeDtypeStruct((B, H, W), jnp.int32),
            jax.ShapeDtypeStruct((4, B, H, W), jnp.float32),
        ],
    )(tenFlow, tenMetric)

    flags = pl.pallas_call(
        functools.partial(_flag_body, H, W, 4),
        out_shape=jax.ShapeDtypeStruct((B, H // 8, 4), jnp.int32),
    )(t)

    splat = _make_splat(B, C, H, W)
    out_flat, _ = splat(tenInput.reshape(B * C * HW),
                        t.reshape(B * HW),
                        wm.reshape(4 * B * HW),
                        flags.reshape(B * (H // 8) * 4))
    return out_flat.reshape(B, C, H, W)


# speedup vs baseline: 1.5692x; 1.0010x over previous
import functools

import jax
import jax.numpy as jnp
from jax import lax
from jax.experimental import pallas as pl
from jax.experimental.pallas import tpu as pltpu
from jax.experimental.pallas import tpu_sc as plsc


def _prep_body(H, W, flow_ref, metric_ref, t_ref, wm_ref):
    i = pl.program_id(1)
    x = lax.broadcasted_iota(jnp.int32, (8, W), 1).astype(jnp.float32)
    y = (lax.broadcasted_iota(jnp.int32, (8, W), 0) + i * 8).astype(jnp.float32)
    fx = x + flow_ref[0, 0]
    fy = y + flow_ref[0, 1]
    x0f = jnp.floor(fx)
    y0f = jnp.floor(fy)
    x1f = x0f + 1.0
    y1f = y0f + 1.0
    m = jnp.exp(metric_ref[0, 0])
    wnw = (x1f - fx) * (y1f - fy)
    wne = (fx - x0f) * (y1f - fy)
    wsw = (x1f - fx) * (fy - y0f)
    wse = (fx - x0f) * (fy - y0f)
    mx0 = (x0f >= 0.0) & (x0f <= W - 1.0)
    mx1 = (x1f >= 0.0) & (x1f <= W - 1.0)
    my0 = (y0f >= 0.0) & (y0f <= H - 1.0)
    my1 = (y1f >= 0.0) & (y1f <= H - 1.0)
    zero = jnp.float32(0.0)
    wm_ref[0, 0] = jnp.where(mx0 & my0, wnw, zero) * m
    wm_ref[1, 0] = jnp.where(mx1 & my0, wne, zero) * m
    wm_ref[2, 0] = jnp.where(mx0 & my1, wsw, zero) * m
    wm_ref[3, 0] = jnp.where(mx1 & my1, wse, zero) * m
    ex = jnp.clip(x0f, -1.0, W - 1.0) + 1.0
    ey = jnp.clip(y0f, -1.0, H - 1.0) + 1.0
    t_ref[0] = (ey * W + ex).astype(jnp.int32)


def _flag_body(H, W, NSEG, t_ref, flag_ref):
    B = t_ref.shape[0]
    HW = H * W
    SEG = HW // NSEG
    NW = H // 8
    t = t_ref[...].reshape(B * NW, 8 * W)
    tmin = jnp.min(t, axis=1)
    tmax = jnp.max(t, axis=1)
    cols = []
    for h in range(NSEG):
        hit = (tmax >= h * SEG) & (tmin <= h * SEG + SEG + W)
        cols.append(hit.astype(jnp.int32))
    flag_ref[...] = jnp.stack(cols, axis=1).reshape(B, NW, NSEG)


def _make_splat(B, C, H, W, interpret=False):
    HW = H * W
    P = 8 * W
    QTR = HW // 4
    NWIN = HW // P
    NTASK = 4 + 4 * (C // 2)
    NROUND = (NTASK + 15) // 16
    OFFS = (W + 1, W, 1, 0)
    assert C % 2 == 0 and HW % P == 0 and QTR % P == 0 and NWIN % 2 == 0

    mesh = plsc.VectorSubcoreMesh(core_axis_name="c", subcore_axis_name="s")

    @functools.partial(
        pl.kernel,
        out_type=(jax.ShapeDtypeStruct((B * C * HW,), jnp.float32),
                  jax.ShapeDtypeStruct((B * HW,), jnp.float32)),
        mesh=mesh,
        scratch_types=[
            pltpu.VMEM((QTR,), jnp.float32),
            pltpu.VMEM((QTR,), jnp.float32),
            pltpu.VMEM((2, P), jnp.float32),
            pltpu.VMEM((2, P), jnp.float32),
            pltpu.VMEM((2, P), jnp.int32),
            pltpu.VMEM((2, P), jnp.float32),
            pltpu.VMEM((2, P), jnp.float32),
            pltpu.VMEM((2, P), jnp.float32),
            pltpu.VMEM((2, P), jnp.float32),
            pltpu.VMEM((4 * HW // P + 8,), jnp.int32),
            pltpu.SemaphoreType.DMA,
            pltpu.SemaphoreType.DMA,
            pltpu.SemaphoreType.DMA,
            pltpu.SemaphoreType.DMA,
        ],
        compiler_params=pltpu.CompilerParams(needs_layout_passes=False),
        interpret=interpret,
    )
    def splat(inp_hbm, t_hbm, wm_hbm, flags_hbm, out_hbm, rn_hbm,
              acc_a, acc_b, inp_a, inp_b, t_b, w0, w1, w2, w3, flag_buf,
              sem0, sem1, semo0, semo1):
        b = lax.axis_index("c")
        s = lax.axis_index("s")
        wbufs = (w0, w1, w2, w3)
        ibufs = (inp_a, inp_b)
        accs = (acc_a, acc_b)
        sems = (sem0, sem1)
        semo = (semo0, semo1)
        pltpu.sync_copy(
            flags_hbm.at[pl.ds(pl.multiple_of(b * 4 * NWIN, 8), 4 * NWIN)],
            flag_buf.at[pl.ds(0, 4 * NWIN)])

        def wflag(wi, q):
            gat = plsc.load_gather(
                flag_buf, [jnp.full((16,), wi * 4 + q, jnp.int32)])
            return jnp.max(gat) > 0

        def accumulate(c0, q, nch):
            nacc = max(nch, 1)
            base = q * QTR

            for ci in range(nacc):
                @plsc.parallel_loop(0, QTR // 16, unroll=8)
                def _(i):
                    accs[ci][pl.ds(pl.multiple_of(i * 16, 16), 16)] = (
                        jnp.zeros((16,), jnp.float32))

            def issue(wi, pb):
                p0 = wi * P
                for ci in range(nch):
                    pltpu.async_copy(
                        inp_hbm.at[pl.ds(
                            pl.multiple_of((b * C + c0 + ci) * HW + p0, 8), P)],
                        ibufs[ci].at[pb], sems[pb])
                pltpu.async_copy(
                    t_hbm.at[pl.ds(pl.multiple_of(b * HW + p0, 8), P)],
                    t_b.at[pb], sems[pb])
                for k in range(4):
                    pltpu.async_copy(
                        wm_hbm.at[pl.ds(pl.multiple_of((k * B + b) * HW + p0, 8), P)],
                        wbufs[k].at[pb], sems[pb])

            def wait(pb):
                for ci in range(nch):
                    pltpu.make_async_copy(
                        inp_hbm.at[pl.ds(0, P)], ibufs[ci].at[pb], sems[pb]).wait()
                pltpu.make_async_copy(
                    t_hbm.at[pl.ds(0, P)], t_b.at[pb], sems[pb]).wait()
                for k in range(4):
                    pltpu.make_async_copy(
                        wm_hbm.at[pl.ds(0, P)], wbufs[k].at[pb], sems[pb]).wait()

            lim = jnp.uint32(QTR)

            def compute(pb):
                @plsc.parallel_loop(0, P // 16, unroll=8)
                def _(g):
                    s16 = pl.ds(pl.multiple_of(g * 16, 16), 16)
                    tv = t_b[pb, s16]
                    ivs = [ibufs[ci][pb, s16] for ci in range(nch)]
                    for k in range(4):
                        wv = wbufs[k][pb, s16]
                        loc = tv - (base + OFFS[k])
                        msk = plsc.bitcast(loc, jnp.uint32) < lim
                        if nch == 0:
                            plsc.addupdate_scatter(accs[0], [loc], wv, mask=msk)
                        else:
                            for ci in range(nch):
                                plsc.addupdate_scatter(
                                    accs[ci], [loc], ivs[ci] * wv, mask=msk)

            for pb in range(2):
                @pl.when(wflag(pb, q))
                def _():
                    issue(pb, pb)

            def win2(j, _):
                for pb in range(2):
                    wi = j * 2 + pb

                    @pl.when(wflag(wi, q))
                    def _():
                        wait(pb)
                        compute(pb)

                    @pl.when((wi + 2 < NWIN) & wflag(wi + 2, q))
                    def _():
                        issue(wi + 2, pb)
                return 0
            lax.fori_loop(0, NWIN // 2, win2, 0)

        obufs = (w1, w2)

        def flush(c0, q):
            NCH = QTR // P

            def rb_issue(qi, pb):
                pltpu.async_copy(
                    rn_hbm.at[pl.ds(pl.multiple_of(b * HW + q * QTR + qi * P, 8), P)],
                    w0.at[pb], sems[pb])

            rb_issue(0, 0)
            rb_issue(1, 1)

            def chunk2(j, _):
                for pb in range(2):
                    qi = j * 2 + pb
                    q0 = qi * P
                    pltpu.make_async_copy(
                        rn_hbm.at[pl.ds(0, P)], w0.at[pb], sems[pb]).wait()

                    @pl.when(qi >= 2)
                    def _():
                        for ci in range(2):
                            pltpu.make_async_copy(
                                obufs[ci].at[pb],
                                out_hbm.at[pl.ds(0, P)], semo[pb]).wait()

                    for ci in range(2):
                        @plsc.parallel_loop(0, P // 16, unroll=4)
                        def _(g):
                            s16 = pl.ds(pl.multiple_of(g * 16, 16), 16)
                            a16 = pl.ds(pl.multiple_of(q0 + g * 16, 16), 16)
                            obufs[ci][pb, s16] = accs[ci][a16] * w0[pb, s16]
                        pltpu.async_copy(
                            obufs[ci].at[pb],
                            out_hbm.at[pl.ds(pl.multiple_of(
                                (b * C + c0 + ci) * HW + q * QTR + q0, 8), P)],
                            semo[pb])

                    @pl.when(qi + 2 < NCH)
                    def _():
                        rb_issue(qi + 2, pb)
                return 0
            lax.fori_loop(0, NCH // 2, chunk2, 0)
            for pb in range(2):
                for ci in range(2):
                    pltpu.make_async_copy(
                        obufs[ci].at[pb], out_hbm.at[pl.ds(0, P)], semo[pb]).wait()

        def norm_finalize(q):
            NCH = QTR // P

            def chunk2(j, _):
                for pb in range(2):
                    qi = j * 2 + pb
                    q0 = qi * P

                    @pl.when(qi >= 2)
                    def _():
                        pltpu.make_async_copy(
                            w1.at[pb], rn_hbm.at[pl.ds(0, P)], semo[pb]).wait()

                    @plsc.parallel_loop(0, P // 16, unroll=4)
                    def _(g):
                        s16 = pl.ds(pl.multiple_of(g * 16, 16), 16)
                        a16 = pl.ds(pl.multiple_of(q0 + g * 16, 16), 16)
                        n = acc_a[a16]
                        d = jnp.where(n == 0.0, jnp.float32(1.0), n)
                        w1[pb, s16] = jnp.float32(1.0) / d
                    pltpu.async_copy(
                        w1.at[pb],
                        rn_hbm.at[pl.ds(pl.multiple_of(b * HW + q * QTR + q0, 8), P)],
                        semo[pb])
                return 0
            lax.fori_loop(0, NCH // 2, chunk2, 0)
            for pb in range(2):
                pltpu.make_async_copy(
                    w1.at[pb], rn_hbm.at[pl.ds(0, P)], semo[pb]).wait()

        ct0 = s - 4
        c00 = (ct0 // 4) * 2
        q0_ = ct0 % 4

        @pl.when(s < 4)
        def _():
            accumulate(0, s, nch=0)
            norm_finalize(s)

        @pl.when(s >= 4)
        def _():
            accumulate(c00, q0_, nch=2)

        plsc.subcore_barrier()

        @pl.when(s >= 4)
        def _():
            flush(c00, q0_)

        def round_body(r, _):
            task = r * 16 + s
            ct = task - 4
            c0 = (ct // 4) * 2
            q = ct % 4

            @pl.when(task < NTASK)
            def _():
                accumulate(c0, q, nch=2)
                flush(c0, q)
            return 0
        lax.fori_loop(1, NROUND, round_body, 0)

    return splat


def kernel(tenInput, tenFlow, tenMetric):
    B, C, H, W = tenInput.shape
    HW = H * W

    t, wm = pl.pallas_call(
        functools.partial(_prep_body, H, W),
        grid=(B, H // 8),
        in_specs=[
            pl.BlockSpec((1, 2, 8, W), lambda b, i: (b, 0, i, 0)),
            pl.BlockSpec((1, 1, 8, W), lambda b, i: (b, 0, i, 0)),
        ],
        out_specs=[
            pl.BlockSpec((1, 8, W), lambda b, i: (b, i, 0)),
            pl.BlockSpec((4, 1, 8, W), lambda b, i: (0, b, i, 0)),
        ],
        out_shape=[
            jax.ShapeDtypeStruct((B, H, W), jnp.int32),
            jax.ShapeDtypeStruct((4, B, H, W), jnp.float32),
        ],
    )(tenFlow, tenMetric)

    flags = pl.pallas_call(
        functools.partial(_flag_body, H, W, 4),
        out_shape=jax.ShapeDtypeStruct((B, H // 8, 4), jnp.int32),
    )(t)

    splat = _make_splat(B, C, H, W)
    out_flat, _ = splat(tenInput.reshape(B * C * HW),
                        t.reshape(B * HW),
                        wm.reshape(4 * B * HW),
                        flags.reshape(B * (H // 8) * 4))
    return out_flat.reshape(B, C, H, W)
